# baseline (speedup 1.0000x reference)
import functools

import jax
import jax.numpy as jnp
from jax import lax
from jax.experimental import pallas as pl
from jax.experimental.pallas import tpu as pltpu
from jax.experimental.pallas import tpu_sc as plsc

L = 16
NS = 16


@functools.lru_cache(maxsize=None)
def _build(B: int, V: int):
    assert B % (NS * L) == 0
    bpw = B // NS

    mesh = plsc.VectorSubcoreMesh(
        core_axis_name="c", subcore_axis_name="s", num_cores=1
    )

    @functools.partial(
        pl.kernel,
        out_type=jax.ShapeDtypeStruct((L,), jnp.float32),
        mesh=mesh,
        scratch_types=[
            pltpu.VMEM((bpw,), jnp.int32),
            pltpu.VMEM((bpw,), jnp.int32),
            pltpu.VMEM((bpw,), jnp.float32),
            pltpu.VMEM((L,), jnp.float32),
            pltpu.VMEM_SHARED((NS, L), jnp.float32),
            pltpu.VMEM((NS, L), jnp.float32),
            pltpu.SemaphoreType.DMA,
        ],
    )
    def gather_mean(table, yref, out, y_v, idx_v, g_v, acc_v, shared, red_v, sem):
        sid = lax.axis_index("s")
        base = sid * bpw
        pltpu.sync_copy(yref.at[pl.ds(base, bpw)], y_v)
        iota = lax.iota(jnp.int32, L)
        for c in range(bpw // L):
            chunk = y_v[pl.ds(c * L, L)]
            idx_v[pl.ds(c * L, L)] = (base + c * L + iota) * V + chunk
        pltpu.async_copy(table.at[idx_v], g_v, sem).wait()
        acc = jnp.zeros((L,), jnp.float32)
        for c in range(bpw // L):
            acc = acc + g_v[pl.ds(c * L, L)]
        acc_v[...] = acc
        pltpu.sync_copy(acc_v, shared.at[sid])
        plsc.subcore_barrier()

        @pl.when(sid == 0)
        def _():
            pltpu.sync_copy(shared, red_v)
            tot = red_v[0]
            for j in range(1, NS):
                tot = tot + red_v[j]
            s = tot[0]
            for j in range(1, L):
                s = s + tot[j]
            acc_v[...] = jnp.broadcast_to(s * (1.0 / B), (L,))
            pltpu.sync_copy(acc_v, out)

    return gather_mean


def kernel(output, y):
    B, V = output.shape
    table = output.reshape(B * V)
    res = _build(B, V)(table, y.astype(jnp.int32))
    return res[0]


# speedup vs baseline: 2.3310x; 2.3310x over previous
import functools

import jax
import jax.numpy as jnp
from jax import lax
from jax.experimental import pallas as pl
from jax.experimental.pallas import tpu as pltpu
from jax.experimental.pallas import tpu_sc as plsc

L = 16
NS = 16


@functools.lru_cache(maxsize=None)
def _build(B: int, V: int):
    assert B % (NS * L) == 0
    bpw = B // NS

    mesh = plsc.VectorSubcoreMesh(
        core_axis_name="c", subcore_axis_name="s", num_cores=1
    )

    @functools.partial(
        pl.kernel,
        out_type=jax.ShapeDtypeStruct((L,), jnp.float32),
        mesh=mesh,
        compiler_params=pltpu.CompilerParams(use_tc_tiling_on_sc=True),
        scratch_types=[
            pltpu.VMEM((bpw,), jnp.int32),
            pltpu.VMEM((bpw, 8, 128), jnp.float32),
            pltpu.VMEM((L,), jnp.float32),
            pltpu.VMEM_SHARED((NS * L,), jnp.float32),
            pltpu.VMEM((NS * L,), jnp.float32),
            pltpu.SemaphoreType.DMA,
        ],
    )
    def gather_mean(out2d, yref, out, y_v, g_v, acc_v, shared, red_v, sem):
        sid = lax.axis_index("s")
        base = sid * bpw
        pltpu.sync_copy(yref.at[pl.ds(base, bpw)], y_v)
        iota = lax.iota(jnp.int32, L)
        ycs = [y_v[pl.ds(c * L, L)] for c in range(bpw // L)]
        copies = []
        for j in range(bpw):
            yj = ycs[j // L][j % L]
            col = pl.multiple_of(lax.shift_left(lax.shift_right_logical(yj, 7), 7), 128)
            row = pl.multiple_of(base + (j // 8) * 8, 8)
            copies.append(
                pltpu.async_copy(
                    out2d.at[pl.ds(row, 8), pl.ds(col, 128)],
                    g_v.at[j],
                    sem,
                )
            )
        for cp in copies:
            cp.wait()
        acc = jnp.zeros((L,), jnp.float32)
        for j in range(bpw):
            lane = lax.bitwise_and(ycs[j // L][j % L], 127)
            for k in range(128 // L):
                chunk = g_v[j, j % 8, pl.ds(k * L, L)]
                acc = acc + jnp.where(iota + (k * L) == lane, chunk, 0.0)
        acc_v[...] = acc
        pltpu.sync_copy(acc_v, shared.at[pl.ds(sid * L, L)])
        plsc.subcore_barrier()

        @pl.when(sid == 0)
        def _():
            pltpu.sync_copy(shared, red_v)
            tot = red_v[pl.ds(0, L)]
            for j in range(1, NS):
                tot = tot + red_v[pl.ds(j * L, L)]
            s = tot[0]
            for j in range(1, L):
                s = s + tot[j]
            acc_v[...] = jnp.broadcast_to(s * (1.0 / B), (L,))
            pltpu.sync_copy(acc_v, out)

    return gather_mean


def kernel(output, y):
    B, V = output.shape
    res = _build(B, V)(output, y.astype(jnp.int32))
    return res[0]


# speedup vs baseline: 36.7124x; 15.7496x over previous
import functools

import jax
import jax.numpy as jnp
from jax import lax
from jax.experimental import pallas as pl
from jax.experimental.pallas import tpu as pltpu
from jax.experimental.pallas import tpu_sc as plsc

L = 16
NS = 16


@functools.lru_cache(maxsize=None)
def _build(B: int, V: int):
    assert B % (NS * L) == 0 and V % 8 == 0
    bpw = B // NS

    mesh = plsc.VectorSubcoreMesh(
        core_axis_name="c", subcore_axis_name="s", num_cores=1
    )

    @functools.partial(
        pl.kernel,
        out_type=jax.ShapeDtypeStruct((L,), jnp.float32),
        mesh=mesh,
        compiler_params=pltpu.CompilerParams(use_tc_tiling_on_sc=True),
        scratch_types=[
            pltpu.VMEM((bpw,), jnp.int32),
            pltpu.VMEM((bpw, 8, 128), jnp.float32),
            pltpu.VMEM((L,), jnp.float32),
            pltpu.VMEM_SHARED((NS * L,), jnp.float32),
            pltpu.VMEM((NS * L,), jnp.float32),
            pltpu.SemaphoreType.DMA,
        ],
    )
    def gather_mean(tvb, yref, out, y_v, g_v, acc_v, shared, red_v, sem):
        sid = lax.axis_index("s")
        base = sid * bpw
        pltpu.sync_copy(yref.at[pl.ds(base, bpw)], y_v)
        iota = lax.iota(jnp.int32, L)
        ycs = [y_v[pl.ds(c * L, L)] for c in range(bpw // L)]
        copies = []
        for j in range(bpw):
            yj = ycs[j // L][j % L]
            vrow = pl.multiple_of(
                lax.shift_left(lax.shift_right_logical(yj, 3), 3), 8
            )
            cb = pl.multiple_of(
                lax.shift_left(lax.shift_right_logical(base + j, 7), 7), 128
            )
            copies.append(
                pltpu.async_copy(
                    tvb.at[pl.ds(vrow, 8), pl.ds(cb, 128)], g_v.at[j], sem
                )
            )
        for cp in copies:
            cp.wait()
        acc = jnp.zeros((L,), jnp.float32)
        for j in range(bpw):
            yj = ycs[j // L][j % L]
            sub = lax.bitwise_and(yj, 7)
            lane = lax.bitwise_and(base + j, 127)
            lane_c = lax.shift_left(lax.shift_right_logical(lane, 4), 4)
            chunk = g_v[j, sub, pl.ds(lane_c, L)]
            acc = acc + jnp.where(iota == lax.bitwise_and(lane, L - 1), chunk, 0.0)
        acc_v[...] = acc
        pltpu.sync_copy(acc_v, shared.at[pl.ds(sid * L, L)])
        plsc.subcore_barrier()

        @pl.when(sid == 0)
        def _():
            pltpu.sync_copy(shared, red_v)
            tot = red_v[pl.ds(0, L)]
            for j in range(1, NS):
                tot = tot + red_v[pl.ds(j * L, L)]
            s = tot[0]
            for j in range(1, L):
                s = s + tot[j]
            acc_v[...] = jnp.broadcast_to(s * (1.0 / B), (L,))
            pltpu.sync_copy(acc_v, out)

    return gather_mean


def kernel(output, y):
    B, V = output.shape
    res = _build(B, V)(output.T, y.astype(jnp.int32))
    return res[0]


# speedup vs baseline: 43.4184x; 1.1827x over previous
import functools

import jax
import jax.numpy as jnp
from jax import lax
from jax.experimental import pallas as pl
from jax.experimental.pallas import tpu as pltpu
from jax.experimental.pallas import tpu_sc as plsc

L = 16
NS = 16


@functools.lru_cache(maxsize=None)
def _build(B: int, V: int):
    assert B % (NS * L) == 0 and V % 8 == 0 and B % 128 == 0
    bpw = B // NS
    nlines = (V * B) // 128

    mesh = plsc.VectorSubcoreMesh(
        core_axis_name="c", subcore_axis_name="s", num_cores=1
    )

    @functools.partial(
        pl.kernel,
        out_type=jax.ShapeDtypeStruct((L,), jnp.float32),
        mesh=mesh,
        compiler_params=pltpu.CompilerParams(use_tc_tiling_on_sc=True),
        scratch_types=[
            pltpu.VMEM((bpw,), jnp.int32),
            pltpu.VMEM((bpw,), jnp.int32),
            pltpu.VMEM((bpw, 128), jnp.float32),
            pltpu.VMEM((L,), jnp.float32),
            pltpu.VMEM_SHARED((NS * L,), jnp.float32),
            pltpu.VMEM((NS * L,), jnp.float32),
            pltpu.SemaphoreType.DMA,
        ],
    )
    def gather_mean(lines, yref, out, y_v, idx_v, g_v, acc_v, shared, red_v, sem):
        sid = lax.axis_index("s")
        base = sid * bpw
        pltpu.sync_copy(yref.at[pl.ds(base, bpw)], y_v)
        iota = lax.iota(jnp.int32, L)
        bblock = lax.shift_left(lax.shift_right_logical(base, 7), 3)
        for c in range(bpw // L):
            yc = y_v[pl.ds(c * L, L)]
            line = (
                lax.shift_left(lax.shift_right_logical(yc, 3), 6)
                + bblock
                + lax.bitwise_and(yc, 7)
            )
            idx_v[pl.ds(c * L, L)] = line
        pltpu.async_copy(lines.at[idx_v], g_v, sem).wait()
        acc = jnp.zeros((L,), jnp.float32)
        for j in range(bpw):
            lane = lax.bitwise_and(base + j, 127)
            lane_c = lax.shift_left(lax.shift_right_logical(lane, 4), 4)
            chunk = g_v[j, pl.ds(lane_c, L)]
            acc = acc + jnp.where(iota == lax.bitwise_and(lane, L - 1), chunk, 0.0)
        acc_v[...] = acc
        pltpu.sync_copy(acc_v, shared.at[pl.ds(sid * L, L)])
        plsc.subcore_barrier()

        @pl.when(sid == 0)
        def _():
            pltpu.sync_copy(shared, red_v)
            tot = red_v[pl.ds(0, L)]
            for j in range(1, NS):
                tot = tot + red_v[pl.ds(j * L, L)]
            s = tot[0]
            for j in range(1, L):
                s = s + tot[j]
            acc_v[...] = jnp.broadcast_to(s * (1.0 / B), (L,))
            pltpu.sync_copy(acc_v, out)

    return gather_mean


def kernel(output, y):
    B, V = output.shape
    lines = (
        output.T.reshape(V // 8, 8, B // 128, 128)
        .transpose(0, 2, 1, 3)
        .reshape((V * B) // 128, 128)
    )
    res = _build(B, V)(lines, y.astype(jnp.int32))
    return res[0]


# speedup vs baseline: 43.5920x; 1.0040x over previous
import functools

import jax
import jax.numpy as jnp
from jax import lax
from jax.experimental import pallas as pl
from jax.experimental.pallas import tpu as pltpu
from jax.experimental.pallas import tpu_sc as plsc

L = 16
NS = 16


@functools.lru_cache(maxsize=None)
def _build(B: int, V: int):
    assert B % (NS * L) == 0 and V % 8 == 0 and B % 128 == 0
    bpw = B // NS

    mesh = plsc.VectorSubcoreMesh(
        core_axis_name="c", subcore_axis_name="s", num_cores=1
    )

    @functools.partial(
        pl.kernel,
        out_type=jax.ShapeDtypeStruct((L,), jnp.float32),
        mesh=mesh,
        compiler_params=pltpu.CompilerParams(use_tc_tiling_on_sc=True),
        scratch_types=[
            pltpu.VMEM((bpw,), jnp.int32),
            pltpu.VMEM((bpw,), jnp.int32),
            pltpu.VMEM((bpw, 128), jnp.float32),
            pltpu.VMEM((L,), jnp.float32),
            pltpu.VMEM_SHARED((NS * L,), jnp.float32),
            pltpu.VMEM((NS * L,), jnp.float32),
            pltpu.SemaphoreType.DMA,
        ],
    )
    def gather_mean(lines, yref, out, y_v, idx_v, g_v, acc_v, shared, red_v, sem):
        sid = lax.axis_index("s")
        base = sid * bpw
        pltpu.sync_copy(yref.at[pl.ds(base, bpw)], y_v)
        iota = lax.iota(jnp.int32, L)
        bblock = lax.shift_left(lax.shift_right_logical(base, 7), 3)
        for c in range(bpw // L):
            yc = y_v[pl.ds(c * L, L)]
            line = (
                lax.shift_left(lax.shift_right_logical(yc, 3), 6)
                + bblock
                + lax.bitwise_and(yc, 7)
            )
            idx_v[pl.ds(c * L, L)] = line
        pltpu.async_copy(lines.at[idx_v], g_v, sem).wait()
        acc = jnp.zeros((L,), jnp.float32)
        for j in range(bpw):
            lane = lax.bitwise_and(base + j, 127)
            lane_c = lax.shift_left(lax.shift_right_logical(lane, 4), 4)
            chunk = g_v[j, pl.ds(lane_c, L)]
            acc = acc + jnp.where(iota == lax.bitwise_and(lane, L - 1), chunk, 0.0)
        acc_v[...] = acc
        pltpu.sync_copy(acc_v, shared.at[pl.ds(sid * L, L)])
        plsc.subcore_barrier()

        @pl.when(sid == 0)
        def _():
            pltpu.sync_copy(shared, red_v)
            tot = red_v[pl.ds(0, L)]
            for j in range(1, NS):
                tot = tot + red_v[pl.ds(j * L, L)]
            s = tot[0]
            for j in range(1, L):
                s = s + tot[j]
            acc_v[...] = jnp.broadcast_to(s * (1.0 / B), (L,))
            pltpu.sync_copy(acc_v, out)

    return gather_mean


def kernel(output, y):
    B, V = output.shape
    lines = (
        output.T.reshape(V // 8, 8, B // 128, 128)
        .transpose(0, 2, 1, 3)
        .reshape((V * B) // 128, 128)
    )
    res = _build(B, V)(lines, y.astype(jnp.int32))
    return res[0]
